# baseline (speedup 1.0000x reference)
import functools

import jax
import jax.numpy as jnp
from jax import lax
from jax.experimental import pallas as pl
from jax.experimental.pallas import tpu as pltpu
from jax.experimental.pallas import tpu_sc as plsc

NC, NS, L = 2, 16, 16
NW = NC * NS


@functools.cache
def _build(B, C, H, W):
    HW = H * W
    D2 = C // 2
    BPW = B // NW
    J = 2 * W
    NK = HW // J
    JP = J + 1
    NCV = C // L

    mesh = plsc.VectorSubcoreMesh(core_axis_name="c", subcore_axis_name="s")

    @functools.partial(
        pl.kernel,
        out_type=jax.ShapeDtypeStruct((B, HW, C), jnp.float32),
        mesh=mesh,
        scratch_types=[
            pltpu.VMEM((C, JP), jnp.float32),
            pltpu.VMEM((J, C), jnp.float32),
            pltpu.VMEM((H, D2), jnp.float32),
            pltpu.VMEM((W, D2), jnp.float32),
        ],
        compiler_params=pltpu.CompilerParams(
            use_tc_tiling_on_sc=False, needs_layout_passes=False),
    )
    def pos_add(in_hbm, row_hbm, col_hbm, out_hbm, in_buf, out_buf, row_buf, col_buf):
        wid = lax.axis_index("s") * NC + lax.axis_index("c")
        pltpu.sync_copy(row_hbm.at[pl.ds(0, H), :], row_buf)
        pltpu.sync_copy(col_hbm.at[pl.ds(0, W), :], col_buf)
        lanes = lax.broadcasted_iota(jnp.int32, (L,), 0)

        def chunk(t, _):
            b = wid * BPW + t // NK
            k = t % NK
            pltpu.sync_copy(in_hbm.at[b, :, pl.ds(k * J, J)],
                            in_buf.at[:, pl.ds(0, J)])

            def body(j, _):
                w = lax.bitwise_and(j, W - 1)
                h = (J // W) * k + lax.shift_right_logical(j, 5)
                jvec = jnp.full((L,), j, jnp.int32)
                for cv in range(NCV):
                    rows = cv * L + lanes
                    x = plsc.load_gather(in_buf, [rows, jvec])
                    if cv < NCV // 2:
                        p = col_buf[w, pl.ds(cv * L, L)]
                    else:
                        p = row_buf[h, pl.ds((cv - NCV // 2) * L, L)]
                    out_buf[j, pl.ds(cv * L, L)] = x + p
                return 0

            lax.fori_loop(0, J, body, 0)
            pltpu.sync_copy(out_buf, out_hbm.at[b, pl.ds(k * J, J), :])
            return 0

        lax.fori_loop(0, BPW * NK, chunk, 0)

    return pos_add


def kernel(input, row_embed, col_embed):
    B, C, H, W = input.shape
    inp = input.reshape(B, C, H * W)
    return _build(B, C, H, W)(inp, row_embed, col_embed)


# speedup vs baseline: 1.1291x; 1.1291x over previous
import functools

import jax
import jax.numpy as jnp
from jax import lax
from jax.experimental import pallas as pl
from jax.experimental.pallas import tpu as pltpu
from jax.experimental.pallas import tpu_sc as plsc

NC, NS, L = 2, 16, 16
NW = NC * NS


@functools.cache
def _build(B, C, H, W):
    HW = H * W
    D2 = C // 2
    BPW = B // NW
    J = 2 * W
    NK = HW // J
    CP = C + 1
    WP = D2 + 1
    NQ = J // L
    NT = BPW * NK

    mesh = plsc.VectorSubcoreMesh(core_axis_name="c", subcore_axis_name="s")

    @functools.partial(
        pl.kernel,
        out_type=jax.ShapeDtypeStruct((B, HW, C), jnp.float32),
        mesh=mesh,
        scratch_types=[
            pltpu.VMEM((2, C, J + 1), jnp.float32),
            pltpu.VMEM((2, J, CP), jnp.float32),
            pltpu.VMEM((H, D2), jnp.float32),
            pltpu.VMEM((W, WP), jnp.float32),
            pltpu.SemaphoreType.DMA,
            pltpu.SemaphoreType.DMA,
            pltpu.SemaphoreType.DMA,
            pltpu.SemaphoreType.DMA,
        ],
        compiler_params=pltpu.CompilerParams(
            use_tc_tiling_on_sc=False, needs_layout_passes=False),
    )
    def pos_add(in_hbm, row_hbm, col_hbm, out_hbm,
                in_buf, out_buf, row_buf, col_pad,
                in_sem0, in_sem1, out_sem0, out_sem1):
        wid = lax.axis_index("s") * NC + lax.axis_index("c")
        in_sems = (in_sem0, in_sem1)
        out_sems = (out_sem0, out_sem1)

        pltpu.sync_copy(row_hbm.at[pl.ds(0, H), :], row_buf)
        pltpu.sync_copy(col_hbm.at[pl.ds(0, W), :], col_pad.at[:, pl.ds(0, D2)])

        lanes = lax.broadcasted_iota(jnp.int32, (L,), 0)
        srows = [q * L + lanes for q in range(NQ)]

        def in_cp(t, ph):
            b = wid * BPW + lax.bitwise_and(t, BPW - 1)
            k = lax.shift_right_logical(t, 1)
            return pltpu.make_async_copy(
                in_hbm.at[b, :, pl.ds(k * J, J)],
                in_buf.at[ph, :, pl.ds(0, J)], in_sems[ph])

        def out_cp(t, ph):
            b = wid * BPW + lax.bitwise_and(t, BPW - 1)
            k = lax.shift_right_logical(t, 1)
            return pltpu.make_async_copy(
                out_buf.at[ph, :, pl.ds(0, C)],
                out_hbm.at[b, pl.ds(k * J, J), :], out_sems[ph])

        in_cp(0, 0).start()

        def step(t2, _):
            for ph in range(2):
                t = t2 * 2 + ph
                nxt = t + 1

                @pl.when(nxt < NT)
                def _():
                    in_cp(nxt, ph ^ 1).start()

                in_cp(t, ph).wait()

                @pl.when(t >= 2)
                def _():
                    out_cp(t - 2, ph).wait()

                in_b = in_buf.at[ph]
                out_b = out_buf.at[ph]
                k = lax.shift_right_logical(t, 1)

                def c_col(c, _):
                    cvec = jnp.full((L,), c, jnp.int32)
                    p0 = plsc.load_gather(col_pad, [lanes, cvec])
                    p1 = plsc.load_gather(col_pad, [lanes + L, cvec])
                    for q in range(NQ):
                        x = in_b[c, pl.ds(q * L, L)]
                        plsc.store_scatter(
                            out_b, [srows[q], cvec], x + (p0, p1)[q % 2])
                    return 0

                lax.fori_loop(0, D2, c_col, 0, unroll=2)

                for hh in range(J // W):
                    h = (J // W) * k + hh
                    ps = [row_buf[h, pl.ds(dv * L, L)] for dv in range(D2 // L)]

                    def j_row(j, _, hh=hh, ps=ps):
                        ja = hh * W + j
                        jvec = jnp.full((L,), ja, jnp.int32)
                        for dv in range(D2 // L):
                            rows = D2 + dv * L + lanes
                            x = plsc.load_gather(in_b, [rows, jvec])
                            out_b[ja, pl.ds(D2 + dv * L, L)] = x + ps[dv]
                        return 0

                    lax.fori_loop(0, W, j_row, 0, unroll=2)

                out_cp(t, ph).start()
            return 0

        lax.fori_loop(0, NT // 2, step, 0)
        out_cp(NT - 2, 0).wait()
        out_cp(NT - 1, 1).wait()

    return pos_add


def kernel(input, row_embed, col_embed):
    B, C, H, W = input.shape
    inp = input.reshape(B, C, H * W)
    return _build(B, C, H, W)(inp, row_embed, col_embed)


# speedup vs baseline: 1.7499x; 1.5498x over previous
import functools

import jax
import jax.numpy as jnp
from jax import lax
from jax.experimental import pallas as pl
from jax.experimental.pallas import tpu as pltpu
from jax.experimental.pallas import tpu_sc as plsc

NC, NS, L = 2, 16, 16
NW = NC * NS


@functools.cache
def _build(B, C, H, W):
    HW = H * W
    D2 = C // 2
    BPW = B // NW
    J = 2 * W
    NK = HW // J
    CP = C + 1
    WP = D2 + 1
    NQ = J // L
    NT = BPW * NK

    mesh = plsc.VectorSubcoreMesh(core_axis_name="c", subcore_axis_name="s")

    @functools.partial(
        pl.kernel,
        out_type=jax.ShapeDtypeStruct((B, HW, C), jnp.float32),
        mesh=mesh,
        scratch_types=[
            pltpu.VMEM((2, C, J + 1), jnp.float32),
            pltpu.VMEM((2, J, CP), jnp.float32),
            pltpu.VMEM((H, D2), jnp.float32),
            pltpu.VMEM((W, WP), jnp.float32),
            pltpu.SemaphoreType.DMA,
            pltpu.SemaphoreType.DMA,
            pltpu.SemaphoreType.DMA,
            pltpu.SemaphoreType.DMA,
        ],
        compiler_params=pltpu.CompilerParams(
            use_tc_tiling_on_sc=False, needs_layout_passes=False),
    )
    def pos_add(in_hbm, row_hbm, col_hbm, out_hbm,
                in_buf, out_buf, row_buf, col_pad,
                in_sem0, in_sem1, out_sem0, out_sem1):
        wid = lax.axis_index("s") * NC + lax.axis_index("c")
        in_sems = (in_sem0, in_sem1)
        out_sems = (out_sem0, out_sem1)

        pltpu.sync_copy(row_hbm.at[pl.ds(0, H), :], row_buf)
        pltpu.sync_copy(col_hbm.at[pl.ds(0, W), :], col_pad.at[:, pl.ds(0, D2)])

        lanes = lax.broadcasted_iota(jnp.int32, (L,), 0)
        srows = [q * L + lanes for q in range(NQ)]

        def in_cp(t, ph):
            b = wid * BPW + lax.bitwise_and(t, BPW - 1)
            k = lax.shift_right_logical(t, 1)
            return pltpu.make_async_copy(
                in_hbm.at[b, :, pl.ds(k * J, J)],
                in_buf.at[ph, :, pl.ds(0, J)], in_sems[ph])

        def out_cp(t, ph):
            b = wid * BPW + lax.bitwise_and(t, BPW - 1)
            k = lax.shift_right_logical(t, 1)
            return pltpu.make_async_copy(
                out_buf.at[ph, :, pl.ds(0, C)],
                out_hbm.at[b, pl.ds(k * J, J), :], out_sems[ph])

        in_cp(0, 0).start()

        def step(t2, _):
            for ph in range(2):
                t = t2 * 2 + ph
                nxt = t + 1

                @pl.when(nxt < NT)
                def _():
                    in_cp(nxt, ph ^ 1).start()

                in_cp(t, ph).wait()

                @pl.when(t >= 2)
                def _():
                    out_cp(t - 2, ph).wait()

                in_b = in_buf.at[ph]
                out_b = out_buf.at[ph]
                k = lax.shift_right_logical(t, 1)

                @plsc.parallel_loop(0, D2, unroll=4)
                def _(c):
                    cvec = jnp.full((L,), c, jnp.int32)
                    p0 = plsc.load_gather(col_pad, [lanes, cvec])
                    p1 = plsc.load_gather(col_pad, [lanes + L, cvec])
                    for q in range(NQ):
                        x = in_b[c, pl.ds(q * L, L)]
                        plsc.store_scatter(
                            out_b, [srows[q], cvec], x + (p0, p1)[q % 2])

                for hh in range(J // W):
                    h = (J // W) * k + hh
                    ps = [row_buf[h, pl.ds(dv * L, L)] for dv in range(D2 // L)]

                    @plsc.parallel_loop(0, W, unroll=4)
                    def _(j, hh=hh, ps=ps):
                        ja = hh * W + j
                        jvec = jnp.full((L,), ja, jnp.int32)
                        for dv in range(D2 // L):
                            rows = D2 + dv * L + lanes
                            x = plsc.load_gather(in_b, [rows, jvec])
                            out_b[ja, pl.ds(D2 + dv * L, L)] = x + ps[dv]

                out_cp(t, ph).start()
            return 0

        lax.fori_loop(0, NT // 2, step, 0)
        out_cp(NT - 2, 0).wait()
        out_cp(NT - 1, 1).wait()

    return pos_add


def kernel(input, row_embed, col_embed):
    B, C, H, W = input.shape
    inp = input.reshape(B, C, H * W)
    return _build(B, C, H, W)(inp, row_embed, col_embed)


# speedup vs baseline: 5.7109x; 3.2636x over previous
import functools

import jax
import jax.numpy as jnp
from jax import lax
from jax.experimental import pallas as pl
from jax.experimental.pallas import tpu as pltpu
from jax.experimental.pallas import tpu_sc as plsc

NC, NS, L = 2, 16, 16
NW = NC * NS


@functools.cache
def _build(B, C, H, W):
    D2 = C // 2
    WT = W // 8
    NCT = C // D2
    R = H * WT * NCT * 8
    RPT = R // NW
    GPT = RPT // 16
    NQ = D2 // L

    mesh = plsc.VectorSubcoreMesh(core_axis_name="c", subcore_axis_name="s")

    @functools.partial(
        pl.kernel,
        out_type=jax.ShapeDtypeStruct((B, R, D2), jnp.float32),
        mesh=mesh,
        scratch_types=[
            pltpu.VMEM((2, RPT, D2), jnp.float32),
            pltpu.VMEM((RPT, D2), jnp.float32),
            pltpu.VMEM((H, D2), jnp.float32),
            pltpu.VMEM((W, D2), jnp.float32),
            pltpu.SemaphoreType.DMA,
            pltpu.SemaphoreType.DMA,
            pltpu.SemaphoreType.DMA,
            pltpu.SemaphoreType.DMA,
        ],
        compiler_params=pltpu.CompilerParams(
            use_tc_tiling_on_sc=False, needs_layout_passes=False),
    )
    def pos_add(in_hbm, row_hbm, col_hbm, out_hbm,
                buf, pos_b, row_buf, col_buf,
                in_sem0, in_sem1, out_sem0, out_sem1):
        wid = lax.axis_index("s") * NC + lax.axis_index("c")
        in_sems = (in_sem0, in_sem1)
        out_sems = (out_sem0, out_sem1)
        r0 = wid * RPT

        pltpu.sync_copy(row_hbm.at[pl.ds(0, H), :], row_buf)
        pltpu.sync_copy(col_hbm.at[pl.ds(0, W), :], col_buf)

        g0 = wid * GPT
        for gg in range(GPT):
            g = g0 + gg
            h = lax.shift_right_logical(g, 2)
            wt = lax.bitwise_and(g, WT - 1)
            for ct in range(NCT):
                for wr in range(8):
                    src = col_buf.at[wt * 8 + wr] if ct == 0 else row_buf.at[h]
                    dst_r = gg * 16 + ct * 8 + wr
                    for q in range(NQ):
                        pos_b[dst_r, pl.ds(q * L, L)] = src[pl.ds(q * L, L)]

        def in_cp(b, ph):
            return pltpu.make_async_copy(
                in_hbm.at[b, pl.ds(r0, RPT), :], buf.at[ph], in_sems[ph])

        def out_cp(b, ph):
            return pltpu.make_async_copy(
                buf.at[ph], out_hbm.at[b, pl.ds(r0, RPT), :], out_sems[ph])

        in_cp(0, 0).start()

        def step(b2, _):
            for ph in range(2):
                b = b2 * 2 + ph

                @pl.when(b + 1 < B)
                def _():
                    in_cp(b + 1, ph ^ 1).start()

                in_cp(b, ph).wait()

                @pl.when(b >= 2)
                def _():
                    out_cp(b - 2, ph).wait()

                data = buf.at[ph]

                @plsc.parallel_loop(0, RPT, unroll=4)
                def _(r):
                    for q in range(NQ):
                        x = data[r, pl.ds(q * L, L)]
                        p = pos_b[r, pl.ds(q * L, L)]
                        data[r, pl.ds(q * L, L)] = x + p

                out_cp(b, ph).start()
            return 0

        lax.fori_loop(0, B // 2, step, 0)
        out_cp(B - 2, 0).wait()
        out_cp(B - 1, 1).wait()

    return pos_add


def kernel(input, row_embed, col_embed):
    B, C, H, W = input.shape
    D2 = C // 2
    x = input.transpose(0, 2, 3, 1)
    x = x.reshape(B, H, W // 8, 8, C // D2, D2)
    x = x.transpose(0, 1, 2, 4, 3, 5)
    x = x.reshape(B, H * (W // 8) * (C // D2) * 8, D2)
    y = _build(B, C, H, W)(x, row_embed, col_embed)
    y = y.reshape(B, H * W // 8, C // D2, 8, D2)
    y = y.transpose(0, 1, 3, 2, 4)
    return y.reshape(B, H * W, C)
